# baseline (speedup 1.0000x reference)
import functools

import jax
import jax.numpy as jnp
from jax import lax
from jax.experimental import pallas as pl
from jax.experimental.pallas import tpu as pltpu
from jax.experimental.pallas import tpu_sc as plsc

B = 16384
D = 64
Y_LOW, Y_HIGH = 14.0, 30.0

_NC = 2
_NS = 16
_NW = _NC * _NS
_CHUNK = B // _NW


def _sc_kernel(sidx_hbm, pidx_hbm, sfac_hbm, sbias_hbm, pfac_hbm, pbias_hbm,
               out_hbm, sidx_v, pidx_v, srows_v, prows_v, sb_v, pb_v, out_v,
               sem):
    wid = lax.axis_index("s") * _NC + lax.axis_index("c")
    base = wid * _CHUNK

    pltpu.sync_copy(sidx_hbm.at[pl.ds(base, _CHUNK)], sidx_v)
    pltpu.sync_copy(pidx_hbm.at[pl.ds(base, _CHUNK)], pidx_v)

    c1 = pltpu.async_copy(sfac_hbm.at[sidx_v], srows_v, sem)
    c2 = pltpu.async_copy(pfac_hbm.at[pidx_v], prows_v, sem)
    c3 = pltpu.async_copy(sbias_hbm.at[sidx_v], sb_v, sem)
    c4 = pltpu.async_copy(pbias_hbm.at[pidx_v], pb_v, sem)
    c1.wait()
    c2.wait()
    c3.wait()
    c4.wait()

    lanes = lax.iota(jnp.int32, 16)
    scale = jnp.full((16,), Y_HIGH - Y_LOW, jnp.float32)
    low = jnp.full((16,), Y_LOW, jnp.float32)

    def group_body(g, _):
        rows = g * 16 + lanes
        acc = sb_v[pl.ds(g * 16, 16)] + pb_v[pl.ds(g * 16, 16)]
        for d in range(D):
            dcol = jnp.full((16,), d, jnp.int32)
            sv = plsc.load_gather(srows_v, [rows, dcol])
            pv = plsc.load_gather(prows_v, [rows, dcol])
            acc = acc + sv * pv
        sig = 1.0 / (1.0 + jnp.exp(-acc))
        out_v[pl.ds(g * 16, 16)] = sig * scale + low
        return 0

    lax.fori_loop(0, _CHUNK // 16, group_body, 0)

    pltpu.sync_copy(out_v, out_hbm.at[pl.ds(base, _CHUNK)])


@jax.jit
def _run(sidx, pidx, sample_factors, sample_bias, peptide_factors,
         peptide_bias):
    mesh = plsc.VectorSubcoreMesh(core_axis_name="c", subcore_axis_name="s")
    f = functools.partial(
        pl.kernel,
        out_type=jax.ShapeDtypeStruct((B,), jnp.float32),
        mesh=mesh,
        compiler_params=pltpu.CompilerParams(use_tc_tiling_on_sc=False,
                                             needs_layout_passes=False),
        scratch_types=[
            pltpu.VMEM((_CHUNK,), jnp.int32),
            pltpu.VMEM((_CHUNK,), jnp.int32),
            pltpu.VMEM((_CHUNK, D), jnp.float32),
            pltpu.VMEM((_CHUNK, D), jnp.float32),
            pltpu.VMEM((_CHUNK,), jnp.float32),
            pltpu.VMEM((_CHUNK,), jnp.float32),
            pltpu.VMEM((_CHUNK,), jnp.float32),
            pltpu.SemaphoreType.DMA,
        ],
    )(_sc_kernel)
    return f(sidx, pidx, sample_factors, sample_bias, peptide_factors,
             peptide_bias)


def kernel(x, sample_factors, sample_bias, peptide_factors, peptide_bias):
    sidx = x[:, 0]
    pidx = x[:, 1]
    res = _run(sidx, pidx, sample_factors, sample_bias.reshape(-1),
               peptide_factors, peptide_bias.reshape(-1))
    return res.reshape(B, 1)
